# baseline (speedup 1.0000x reference)
import functools

import jax
import jax.numpy as jnp
from jax import lax
from jax.experimental import pallas as pl
from jax.experimental.pallas import tpu as pltpu
from jax.experimental.pallas import tpu_sc as plsc

B, C = 4096, 1000
NC, NS, L = 2, 16, 16
NW = NC * NS
ROWS_PER_W = B // NW
CH = 16
NCHUNK = ROWS_PER_W // CH
MARGIN = 1.0
NEG_INF = float("-inf")

RPS = 8
SW = C * RPS
SR = B // RPS
SR_PER_CHUNK = CH // RPS

_mesh = plsc.VectorSubcoreMesh(core_axis_name="c", subcore_axis_name="s")


@functools.partial(
    pl.kernel,
    out_type=jax.ShapeDtypeStruct((NW, L), jnp.float32),
    mesh=_mesh,
    compiler_params=pltpu.CompilerParams(needs_layout_passes=False),
    scratch_types=[
        pltpu.VMEM((SR_PER_CHUNK, SW), jnp.float32),
        pltpu.VMEM((SR_PER_CHUNK, SW), jnp.float32),
        pltpu.VMEM((ROWS_PER_W,), jnp.int32),
        pltpu.VMEM((L,), jnp.float32),
        pltpu.VMEM((L, L), jnp.float32),
        pltpu.SemaphoreType.DMA,
        pltpu.SemaphoreType.DMA,
    ],
)
def _hinge_sc(x_hbm, tgt_hbm, out_hbm, buf0, buf1, tgtv, stage, mscr,
              sem0, sem1):
    cid = lax.axis_index("c")
    sid = lax.axis_index("s")
    wid = sid * NC + cid
    base_sr = wid * (ROWS_PER_W // RPS)

    pltpu.sync_copy(tgt_hbm.at[pl.ds(wid * ROWS_PER_W, ROWS_PER_W)], tgtv)

    bufs = (buf0, buf1)
    sems = (sem0, sem1)
    lane = lax.iota(jnp.int32, L)
    neg_inf_v = lax.broadcast(jnp.float32(NEG_INF), (L,))
    super_v = lax.shift_right_logical(lane, 3)
    subcol_v = jnp.bitwise_and(lane, jnp.int32(RPS - 1)) * C

    copies = [None, None]
    copies[0] = pltpu.async_copy(
        x_hbm.at[pl.ds(base_sr, SR_PER_CHUNK), :], bufs[0], sems[0])

    acc = lax.broadcast(jnp.float32(0.0), (L,))
    for ch in range(NCHUNK):
        par = ch % 2
        copies[par].wait()
        if ch + 1 < NCHUNK:
            npar = (ch + 1) % 2
            copies[npar] = pltpu.async_copy(
                x_hbm.at[pl.ds(base_sr + (ch + 1) * SR_PER_CHUNK,
                               SR_PER_CHUNK), :],
                bufs[npar], sems[npar])
        buf = bufs[par]
        tcol = tgtv[pl.ds(ch * CH, L)]
        gcol = subcol_v + tcol
        pos = plsc.load_gather(buf, [super_v, gcol])
        plsc.store_scatter(buf, [super_v, gcol], neg_inf_v)

        offs = [cc * L for cc in range(C // L)] + [C - L]

        def row_body(r, carry, buf=buf):
            sr = lax.shift_right_logical(r, 3)
            rbase = jnp.bitwise_and(r, jnp.int32(RPS - 1)) * C
            ms = [None] * 4
            for i, off in enumerate(offs):
                v = buf[sr, pl.ds(rbase + off, L)]
                k = i % 4
                ms[k] = v if ms[k] is None else jnp.maximum(ms[k], v)
            m = jnp.maximum(jnp.maximum(ms[0], ms[1]),
                            jnp.maximum(ms[2], ms[3]))
            mscr[r, :] = m
            return carry

        lax.fori_loop(0, CH, row_body, jnp.int32(0))

        rmax = plsc.load_gather(mscr, [lane, jnp.full((L,), 0, jnp.int32)])
        for l in range(1, L):
            col = plsc.load_gather(mscr, [lane, jnp.full((L,), l, jnp.int32)])
            rmax = jnp.maximum(rmax, col)
        acc = acc + jnp.maximum(jnp.float32(0.0),
                                jnp.float32(MARGIN) - pos + rmax)

    stage[...] = acc
    pltpu.sync_copy(stage, out_hbm.at[wid])


def _reduce_tc_body(p_ref, o_ref):
    o_ref[...] = (jnp.sum(p_ref[...]) * jnp.float32(1.0 / B)).reshape(1, 1)


_reduce_tc = pl.pallas_call(
    _reduce_tc_body,
    out_shape=jax.ShapeDtypeStruct((1, 1), jnp.float32),
    in_specs=[pl.BlockSpec(memory_space=pltpu.VMEM)],
    out_specs=pl.BlockSpec(memory_space=pltpu.VMEM),
)


def kernel(input, target):
    xw = input.reshape(SR, SW)
    partials = _hinge_sc(xw, target)
    return _reduce_tc(partials)[0, 0]


# speedup vs baseline: 2.0249x; 2.0249x over previous
import functools

import jax
import jax.numpy as jnp
from jax import lax
from jax.experimental import pallas as pl
from jax.experimental.pallas import tpu as pltpu
from jax.experimental.pallas import tpu_sc as plsc

B, C = 4096, 1000
NC, NS, L = 2, 16, 16
NW = NC * NS
ROWS_PER_W = B // NW
CH = 16
NCHUNK = ROWS_PER_W // CH
MARGIN = 1.0
NEG_INF = float("-inf")

_mesh = plsc.VectorSubcoreMesh(core_axis_name="c", subcore_axis_name="s")


@functools.partial(
    pl.kernel,
    out_type=jax.ShapeDtypeStruct((NW, L), jnp.float32),
    mesh=_mesh,
    compiler_params=pltpu.CompilerParams(needs_layout_passes=False),
    scratch_types=[
        pltpu.VMEM((CH, C), jnp.float32),
        pltpu.VMEM((CH, C), jnp.float32),
        pltpu.VMEM((CH, C), jnp.float32),
        pltpu.VMEM((CH, C), jnp.float32),
        pltpu.VMEM((ROWS_PER_W,), jnp.int32),
        pltpu.VMEM((L,), jnp.float32),
        pltpu.SemaphoreType.DMA,
        pltpu.SemaphoreType.DMA,
        pltpu.SemaphoreType.DMA,
        pltpu.SemaphoreType.DMA,
    ],
)
def _hinge_sc(x_hbm, tgt_hbm, out_hbm, buf0, buf1, buf2, buf3, tgtv, stage,
              sem0, sem1, sem2, sem3):
    cid = lax.axis_index("c")
    sid = lax.axis_index("s")
    wid = sid * NC + cid
    base_row = wid * ROWS_PER_W

    pltpu.sync_copy(tgt_hbm.at[pl.ds(wid * ROWS_PER_W, ROWS_PER_W)], tgtv)

    bufs = (buf0, buf1, buf2, buf3)
    sems = (sem0, sem1, sem2, sem3)
    NBUF = 4
    lane = lax.iota(jnp.int32, L)
    neg_inf_v = lax.broadcast(jnp.float32(NEG_INF), (L,))

    copies = [None] * NBUF
    for p in range(NBUF - 1):
        copies[p] = pltpu.async_copy(
            x_hbm.at[pl.ds(base_row + p * CH, CH), :], bufs[p], sems[p])

    acc = lax.broadcast(jnp.float32(0.0), (L,))
    for ch in range(NCHUNK):
        par = ch % NBUF
        copies[par].wait()
        if ch + NBUF - 1 < NCHUNK:
            npar = (ch + NBUF - 1) % NBUF
            copies[npar] = pltpu.async_copy(
                x_hbm.at[pl.ds(base_row + (ch + NBUF - 1) * CH, CH), :],
                bufs[npar], sems[npar])
        buf = bufs[par]
        tcol = tgtv[pl.ds(ch * CH, L)]
        pos = plsc.load_gather(buf, [lane, tcol])
        plsc.store_scatter(buf, [lane, tcol], neg_inf_v)

        def row_body(r, rmax, buf=buf):
            m = buf[r, pl.ds(0, L)]
            for cc in range(1, C // L):
                m = jnp.maximum(m, buf[r, pl.ds(cc * L, L)])
            m = jnp.maximum(m, buf[r, pl.ds(C - L, L)])
            s = jnp.max(m)
            return jnp.where(lane == r, s, rmax)

        rmax = lax.fori_loop(0, CH, row_body, neg_inf_v)
        acc = acc + jnp.maximum(jnp.float32(0.0),
                                jnp.float32(MARGIN) - pos + rmax)

    stage[...] = acc
    pltpu.sync_copy(stage, out_hbm.at[wid])


def _reduce_tc_body(p_ref, o_ref):
    o_ref[...] = (jnp.sum(p_ref[...]) * jnp.float32(1.0 / B)).reshape(1, 1)


_reduce_tc = pl.pallas_call(
    _reduce_tc_body,
    out_shape=jax.ShapeDtypeStruct((1, 1), jnp.float32),
    in_specs=[pl.BlockSpec(memory_space=pltpu.VMEM)],
    out_specs=pl.BlockSpec(memory_space=pltpu.VMEM),
)


def kernel(input, target):
    partials = _hinge_sc(input, target)
    return _reduce_tc(partials)[0, 0]
